# baseline (speedup 1.0000x reference)
import functools

import jax
import jax.numpy as jnp
from jax.experimental import pallas as pl
from jax.experimental.pallas import tpu as pltpu

N = 4096
D = 128
N_HEADS = 2
KEEP_RATIO = 0.5
INV_TEMP = 2.0
BLK = 256
NB = N // BLK
NEG = -1e9


def _compute_hn(x, w_l, hn_ref):
    for h in range(N_HEADS):
        hh = jnp.dot(x, w_l[h], preferred_element_type=jnp.float32)
        norm = jnp.sqrt(jnp.sum(hh * hh, axis=1, keepdims=True))
        hn_ref[h] = (hh / (norm + 1e-8)).astype(jnp.bfloat16)


def _body(x_ref, adj0_ref, adj1_ref, ori_ref, w_ref, out_ref,
          hn_ref, x1_ref, bias_ref):
    l = pl.program_id(0)
    i = pl.program_id(1)

    @pl.when((l == 0) & (i == 0))
    def _():
        _compute_hn(x_ref[...], w_ref[0], hn_ref)

    @pl.when((l == 1) & (i == 0))
    def _():
        _compute_hn(x1_ref[...], w_ref[1], hn_ref)

    @pl.when(l == 0)
    def _():
        bias_ref[...] = jnp.where(adj0_ref[...] > 0.0, 0.0,
                                  NEG).astype(jnp.bfloat16)

    @pl.when(l == 1)
    def _():
        bias_ref[...] = jnp.where(adj1_ref[...] > 0.0, 0.0,
                                  NEG).astype(jnp.bfloat16)

    bias = bias_ref[...]
    acc = jnp.zeros((BLK, D), jnp.float32)
    for h in range(N_HEADS):
        hn = hn_ref[h]
        hnb = hn_ref[h, pl.ds(i * BLK, BLK), :]
        sim = jax.lax.dot_general(
            hnb, hn, (((1,), (1,)), ((), ())),
            preferred_element_type=jnp.float32)
        sim_m = sim.astype(jnp.bfloat16) + bias
        rmax = jnp.max(sim_m, axis=1, keepdims=True)
        keep = sim_m >= KEEP_RATIO * rmax
        p = jnp.where(keep, jnp.exp((sim_m - rmax) * INV_TEMP),
                      jnp.bfloat16(0.0))
        s = jnp.sum(p.astype(jnp.float32), axis=1, keepdims=True)
        agg = jnp.dot(p, ori_ref[...], preferred_element_type=jnp.float32)
        acc = acc + agg / s
    res = acc * (1.0 / N_HEADS)
    out_ref[0] = res

    @pl.when(l == 0)
    def _():
        x1_ref[pl.ds(i * BLK, BLK), :] = res


@functools.partial(jax.jit, static_argnames=())
def kernel(features, adj0, adj1, W):
    ori_bf = features.astype(jnp.bfloat16)
    out = pl.pallas_call(
        _body,
        grid=(2, NB),
        in_specs=[
            pl.BlockSpec((N, D), lambda l, i: (0, 0)),
            pl.BlockSpec((BLK, N), lambda l, i: (jnp.where(l == 0, i, NB - 1), 0)),
            pl.BlockSpec((BLK, N), lambda l, i: (jnp.where(l == 0, 0, i), 0)),
            pl.BlockSpec((N, D), lambda l, i: (0, 0)),
            pl.BlockSpec((2, N_HEADS, D, D), lambda l, i: (0, 0, 0, 0)),
        ],
        out_specs=pl.BlockSpec((1, BLK, D), lambda l, i: (l, i, 0)),
        out_shape=jax.ShapeDtypeStruct((2, N, D), jnp.float32),
        scratch_shapes=[
            pltpu.VMEM((N_HEADS, N, D), jnp.bfloat16),
            pltpu.VMEM((N, D), jnp.float32),
            pltpu.VMEM((BLK, N), jnp.bfloat16),
        ],
    )(features, adj0, adj1, ori_bf, W)
    return out[1]
